# Initial kernel scaffold; baseline (speedup 1.0000x reference)
import jax
import jax.numpy as jnp
from jax.experimental import pallas as pl


def kernel(z, x, edge_index, norm_factor, batch, W):
    raise NotImplementedError("write your pallas kernel here")


# speedup vs baseline: 13.2507x; 13.2507x over previous
import functools

import jax
import jax.numpy as jnp
from jax import lax
from jax.experimental import pallas as pl
from jax.experimental.pallas import tpu as pltpu
from jax.experimental.pallas import tpu_sc as plsc

_ALPHA = 0.5
_NC = 2
_NS = 16
_L = 16
_B = 128


def _cdiv(a, b):
    return (a + b - 1) // b


def _chunks(total, step):
    out = []
    off = 0
    while off < total:
        out.append((off, min(step, total - off)))
        off += step
    return out


@functools.lru_cache(maxsize=None)
def _build(N, E, D):
    DH = D // 2
    CH = _cdiv(E, _NS * _B)
    BN = N // 10 if N % 10 == 0 and (N // 10) % 8 == 0 else N
    NG = DH // _L
    NP = _cdiv(N, 128) * 128

    def _mm1_body(z_ref, x_ref, nf_ref, w_ref, o_ref):
        zx = z_ref[...] + x_ref[...]
        y = lax.dot_general(zx, w_ref[...], (((1,), (1,)), ((), ())),
                            preferred_element_type=jnp.float32)
        y = y * nf_ref[...]
        o_ref[0] = y[:, :DH]
        o_ref[1] = y[:, DH:]

    mm1 = pl.pallas_call(
        _mm1_body,
        grid=(N // BN,),
        in_specs=[
            pl.BlockSpec((BN, D), lambda i: (i, 0)),
            pl.BlockSpec((BN, D), lambda i: (i, 0)),
            pl.BlockSpec((BN, 1), lambda i: (i, 0)),
            pl.BlockSpec((D, D), lambda i: (0, 0)),
        ],
        out_specs=pl.BlockSpec((2, BN, DH), lambda i: (0, i, 0)),
        out_shape=jax.ShapeDtypeStruct((2, N, DH), jnp.float32),
    )

    orows = NP // _NS

    def _sc_body(table, rows_h, cols_h, out_h,
                 rs, cs, rgj, cgj, Rb, Cb, acc, sem_r, sem_c):
        c = lax.axis_index("c")
        s = lax.axis_index("s")
        cN = c * N

        pltpu.sync_copy(rows_h.at[s], rs)
        pltpu.sync_copy(cols_h.at[s], cs)

        def _zrow(i, carry):
            for g in range(NG):
                Rb[i, pl.ds(g * _L, _L)] = jnp.zeros((_L,), jnp.float32)
            return carry
        lax.fori_loop(0, _B, _zrow, 0)
        zbase = s * orows
        for off, nrows in _chunks(orows, _B):
            pltpu.sync_copy(Rb.at[pl.ds(0, nrows)],
                            acc.at[pl.ds(zbase + off, nrows)])
        plsc.subcore_barrier()

        def _edge(j, carry):
            for g in range(_B // _L):
                sl = pl.ds(g * _L, _L)
                rgj[sl] = rs[j, sl] + cN
                cgj[sl] = cs[j, sl] + cN
            d1 = pltpu.async_copy(table.at[rgj], Rb, sem_r)
            d2 = pltpu.async_copy(table.at[cgj], Cb, sem_c)
            d1.wait()
            d2.wait()

            def _crow(i, icarry):
                for g in range(NG):
                    sl = pl.ds(g * _L, _L)
                    m = jnp.maximum(Rb[i, sl] - Cb[i, sl], 0.0)
                    Rb[i, sl] = m
                    Cb[i, sl] = -m
                return icarry
            lax.fori_loop(0, _B, _crow, 0)

            pltpu.sync_copy(Rb, acc.at[rs.at[j]], add=True)
            pltpu.sync_copy(Cb, acc.at[cs.at[j]], add=True)
            return carry
        lax.fori_loop(0, CH, _edge, 0)
        plsc.subcore_barrier()

        nbase = s * orows
        pltpu.sync_copy(acc.at[pl.ds(nbase, orows)],
                        out_h.at[c, pl.ds(nbase, orows)])

    sc_edge = pl.kernel(
        _sc_body,
        out_type=jax.ShapeDtypeStruct((2, NP, DH), jnp.float32),
        mesh=plsc.VectorSubcoreMesh(core_axis_name="c", subcore_axis_name="s",
                                    num_cores=_NC, num_subcores=_NS),
        compiler_params=pltpu.CompilerParams(use_tc_tiling_on_sc=False),
        scratch_types=[
            pltpu.VMEM((CH, _B), jnp.int32),
            pltpu.VMEM((CH, _B), jnp.int32),
            pltpu.VMEM((_B,), jnp.int32),
            pltpu.VMEM((_B,), jnp.int32),
            pltpu.VMEM((_B, DH), jnp.float32),
            pltpu.VMEM((_B, DH), jnp.float32),
            pltpu.VMEM_SHARED((NP, DH), jnp.float32),
            pltpu.SemaphoreType.DMA,
            pltpu.SemaphoreType.DMA,
        ],
    )

    def _mm2_body(g_ref, z_ref, nf_ref, w_ref, o_ref):
        nf = nf_ref[...]
        y0 = g_ref[0] * nf
        y1 = g_ref[1] * nf
        acc = lax.dot_general(y0, w_ref[...][:DH, :], (((1,), (0,)), ((), ())),
                              preferred_element_type=jnp.float32)
        acc = acc + lax.dot_general(y1, w_ref[...][DH:, :],
                                    (((1,), (0,)), ((), ())),
                                    preferred_element_type=jnp.float32)
        o_ref[...] = (1.0 - _ALPHA) * z_ref[...] - _ALPHA * acc

    mm2 = pl.pallas_call(
        _mm2_body,
        grid=(N // BN,),
        in_specs=[
            pl.BlockSpec((2, BN, DH), lambda i: (0, i, 0)),
            pl.BlockSpec((BN, D), lambda i: (i, 0)),
            pl.BlockSpec((BN, 1), lambda i: (i, 0)),
            pl.BlockSpec((D, D), lambda i: (0, 0)),
        ],
        out_specs=pl.BlockSpec((BN, D), lambda i: (i, 0)),
        out_shape=jax.ShapeDtypeStruct((N, D), jnp.float32),
    )

    return mm1, sc_edge, mm2, CH


def kernel(z, x, edge_index, norm_factor, batch, W):
    del batch
    N, D = z.shape
    E = edge_index.shape[1]
    mm1, sc_edge, mm2, CH = _build(N, E, D)

    a2 = mm1(z, x, norm_factor, W)
    table = a2.reshape(2 * N, D // 2)

    epad = _NS * CH * _B
    rows = jnp.pad(edge_index[0], (0, epad - E)).reshape(_NS, CH, _B)
    cols = jnp.pad(edge_index[1], (0, epad - E)).reshape(_NS, CH, _B)

    g = sc_edge(table, rows, cols)
    return mm2(g, z, norm_factor, W)
